# baseline (speedup 1.0000x reference)
import functools

import jax
import jax.numpy as jnp
from jax import lax
from jax.experimental import pallas as pl
from jax.experimental.pallas import tpu as pltpu
from jax.experimental.pallas import tpu_sc as plsc

NC = 2
NS = 16
NW = NC * NS
CHUNK = 128


@functools.lru_cache(maxsize=None)
def _make_lookup(n_rows: int, hidden: int):
    assert n_rows % (NW * CHUNK) == 0
    bpw = n_rows // NW
    nchunk = bpw // CHUNK
    mesh = plsc.VectorSubcoreMesh(core_axis_name="c", subcore_axis_name="s")

    @functools.partial(
        pl.kernel,
        mesh=mesh,
        out_type=jax.ShapeDtypeStruct((n_rows, hidden), jnp.float32),
        scratch_types=[
            pltpu.VMEM((nchunk, CHUNK), jnp.int32),
            pltpu.VMEM((CHUNK, hidden), jnp.float32),
            pltpu.SemaphoreType.DMA,
        ],
        compiler_params=pltpu.CompilerParams(use_tc_tiling_on_sc=False),
    )
    def lookup(x_hbm, table_hbm, out_hbm, idx_v, rows_v, sem):
        wid = lax.axis_index("s") * NC + lax.axis_index("c")
        base = wid * bpw
        pltpu.sync_copy(x_hbm.at[wid], idx_v)

        def body(j, carry):
            pltpu.async_copy(table_hbm.at[idx_v.at[j]], rows_v, sem).wait()
            pltpu.sync_copy(rows_v, out_hbm.at[pl.ds(base + j * CHUNK, CHUNK)])
            return carry

        lax.fori_loop(0, nchunk, body, 0)

    return lookup


def kernel(x, table):
    n_rows = x.shape[0] * x.shape[1]
    hidden = table.shape[1]
    fn = _make_lookup(n_rows, hidden)
    x3 = x.reshape(NW, n_rows // (NW * CHUNK), CHUNK)
    out = fn(x3, table)
    return out.reshape(x.shape[0], x.shape[1], hidden)


# speedup vs baseline: 1.1159x; 1.1159x over previous
import functools

import jax
import jax.numpy as jnp
from jax import lax
from jax.experimental import pallas as pl
from jax.experimental.pallas import tpu as pltpu
from jax.experimental.pallas import tpu_sc as plsc

NC = 2
NS = 16
NW = NC * NS
CHUNK = 128
G = 4
SUP = G * CHUNK


@functools.lru_cache(maxsize=None)
def _make_lookup(n_rows: int, hidden: int):
    assert n_rows % (NW * 2 * SUP) == 0
    bpw = n_rows // NW
    nchunk = bpw // CHUNK
    nsup = bpw // SUP
    mesh = plsc.VectorSubcoreMesh(core_axis_name="c", subcore_axis_name="s")

    @functools.partial(
        pl.kernel,
        mesh=mesh,
        out_type=jax.ShapeDtypeStruct((n_rows, hidden), jnp.float32),
        scratch_types=[
            pltpu.VMEM((nchunk, CHUNK), jnp.int32),
            pltpu.VMEM((SUP, hidden), jnp.float32),
            pltpu.VMEM((SUP, hidden), jnp.float32),
            pltpu.SemaphoreType.DMA,
            pltpu.SemaphoreType.DMA,
        ],
        compiler_params=pltpu.CompilerParams(use_tc_tiling_on_sc=False),
    )
    def lookup(x_hbm, table_hbm, out_hbm, idx_v, buf0, buf1, sem0, sem1):
        wid = lax.axis_index("s") * NC + lax.axis_index("c")
        base = wid * bpw
        pltpu.sync_copy(x_hbm.at[wid], idx_v)

        def fire(s, buf, sem):
            for i in range(G):
                pltpu.async_copy(
                    table_hbm.at[idx_v.at[s * G + i]],
                    buf.at[pl.ds(i * CHUNK, CHUNK)],
                    sem,
                )

        def drain(buf, sem):
            for i in range(G):
                pltpu.make_async_copy(
                    table_hbm.at[pl.ds(0, CHUNK)],
                    buf.at[pl.ds(i * CHUNK, CHUNK)],
                    sem,
                ).wait()

        def write(s, buf):
            pltpu.sync_copy(buf, out_hbm.at[pl.ds(base + s * SUP, SUP)])

        fire(0, buf0, sem0)
        fire(1, buf1, sem1)

        def body(t, carry):
            s = 2 * t
            drain(buf0, sem0)
            write(s, buf0)
            fire(s + 2, buf0, sem0)
            drain(buf1, sem1)
            write(s + 1, buf1)
            fire(s + 3, buf1, sem1)
            return carry

        lax.fori_loop(0, nsup // 2 - 1, body, 0)
        drain(buf0, sem0)
        write(nsup - 2, buf0)
        drain(buf1, sem1)
        write(nsup - 1, buf1)

    return lookup


def kernel(x, table):
    n_rows = x.shape[0] * x.shape[1]
    hidden = table.shape[1]
    fn = _make_lookup(n_rows, hidden)
    x3 = x.reshape(NW, n_rows // (NW * CHUNK), CHUNK)
    out = fn(x3, table)
    return out.reshape(x.shape[0], x.shape[1], hidden)
